# Initial kernel scaffold; baseline (speedup 1.0000x reference)
import jax
import jax.numpy as jnp
from jax.experimental import pallas as pl


def kernel(intermediate_states, w, router_logits):
    raise NotImplementedError("write your pallas kernel here")


# speedup vs baseline: 1.9004x; 1.9004x over previous
import functools

import jax
import jax.numpy as jnp
from jax import lax
from jax.experimental import pallas as pl
from jax.experimental.pallas import tpu as pltpu
from jax.experimental.pallas import tpu_sc as plsc

NUM_TOKENS = 2048
TOPK = 2
NUM_EXPERTS = 8
HIDDEN = 1024
INTER = 4096
ROWS = NUM_TOKENS * TOPK
BLK = 128
NUM_BLOCKS = ROWS // BLK
NUM_VISITS = NUM_BLOCKS + NUM_EXPERTS - 1

NC = 2
NS = 16
NW = NC * NS
TPW = NUM_TOKENS // NW
CHUNK = 32


def _routing(router_logits):
    probs = jax.nn.softmax(router_logits, axis=-1)
    topk_w, topk_ids = lax.top_k(probs, TOPK)
    flat = topk_ids.reshape(-1).astype(jnp.int32)
    onehot = (flat[:, None] == jnp.arange(NUM_EXPERTS, dtype=jnp.int32)[None, :]).astype(jnp.int32)
    cum = jnp.cumsum(onehot, axis=0)
    counts = cum[-1]
    off = jnp.concatenate([jnp.zeros((1,), jnp.int32),
                           jnp.cumsum(counts).astype(jnp.int32)])
    rank = jnp.take_along_axis(cum - onehot, flat[:, None], axis=1)[:, 0]
    pos = (off[flat] + rank).astype(jnp.int32)
    ws_sorted = jnp.zeros((ROWS,), jnp.float32).at[pos].set(topk_w.reshape(-1))
    idx0 = pos[0::2]
    idx1 = pos[1::2]

    bstart = off[:-1] // BLK
    bend = -((-off[1:]) // BLK)
    nblk = jnp.where(counts > 0, bend - bstart, 0).astype(jnp.int32)
    vstart = jnp.concatenate([jnp.zeros((1,), jnp.int32),
                              jnp.cumsum(nblk).astype(jnp.int32)])
    total = vstart[-1]
    vv = jnp.arange(NUM_VISITS, dtype=jnp.int32)
    e_v = jnp.clip(jnp.searchsorted(vstart, vv, side="right").astype(jnp.int32) - 1,
                   0, NUM_EXPERTS - 1)
    b_v = bstart[e_v] + (vv - vstart[e_v])
    lo_v = jnp.maximum(off[e_v], b_v * BLK)
    hi_v = jnp.minimum(off[e_v + 1], (b_v + 1) * BLK)
    valid = vv < total
    e_pad = e_v[jnp.maximum(total - 1, 0)]
    e_v = jnp.where(valid, e_v, e_pad).astype(jnp.int32)
    b_v = jnp.where(valid, b_v, NUM_BLOCKS - 1).astype(jnp.int32)
    lo_v = jnp.where(valid, lo_v, 0).astype(jnp.int32)
    hi_v = jnp.where(valid, hi_v, 0).astype(jnp.int32)
    return ws_sorted, idx0, idx1, b_v, e_v, lo_v, hi_v


def _gemm_body(vb_ref, ve_ref, lo_ref, hi_ref, x_ref, w_ref, ws_ref, o_ref):
    v = pl.program_id(0)
    b = vb_ref[v]
    lo = lo_ref[v]
    hi = hi_ref[v]
    rows = b * BLK + lax.broadcasted_iota(jnp.int32, (BLK, 1), 0)
    mask = jnp.logical_and(rows >= lo, rows < hi)
    xb = x_ref[...].astype(jnp.bfloat16)
    wb = w_ref[0]
    prod = lax.dot_general(xb, wb, (((1,), (0,)), ((), ())),
                           preferred_element_type=jnp.float32)
    wcol = ws_ref[:, 0:1]
    contrib = prod * jnp.where(mask, wcol, 0.0)
    first = jnp.logical_or(v == 0, vb_ref[jnp.maximum(v - 1, 0)] != b)

    @pl.when(first)
    def _():
        o_ref[...] = contrib

    @pl.when(jnp.logical_not(first))
    def _():
        o_ref[...] += contrib


def _grouped_gemm(x, w_bf, ws_b, b_v, e_v, lo_v, hi_v):
    grid_spec = pltpu.PrefetchScalarGridSpec(
        num_scalar_prefetch=4,
        grid=(NUM_VISITS,),
        in_specs=[
            pl.BlockSpec((BLK, INTER), lambda v, vb, ve, lo, hi: (vb[v], 0)),
            pl.BlockSpec((1, INTER, HIDDEN), lambda v, vb, ve, lo, hi: (ve[v], 0, 0)),
            pl.BlockSpec((BLK, 128), lambda v, vb, ve, lo, hi: (vb[v], 0)),
        ],
        out_specs=pl.BlockSpec((BLK, HIDDEN), lambda v, vb, ve, lo, hi: (vb[v], 0)),
    )
    return pl.pallas_call(
        _gemm_body,
        grid_spec=grid_spec,
        out_shape=jax.ShapeDtypeStruct((ROWS, HIDDEN), jnp.float32),
        compiler_params=pltpu.CompilerParams(
            dimension_semantics=("arbitrary",)),
    )(b_v, e_v, lo_v, hi_v, x, w_bf, ws_b)


def _combine_body(rows_hbm, i0_hbm, i1_hbm, out_hbm, i0c, i1c, r0, r1, sem0, sem1):
    wid = lax.axis_index("s") * NC + lax.axis_index("c")
    base = wid * TPW
    for c in range(TPW // CHUNK):
        tok0 = base + c * CHUNK
        pltpu.sync_copy(i0_hbm.at[pl.ds(tok0, CHUNK)], i0c)
        pltpu.sync_copy(i1_hbm.at[pl.ds(tok0, CHUNK)], i1c)
        g0 = pltpu.async_copy(rows_hbm.at[i0c], r0, sem0)
        g1 = pltpu.async_copy(rows_hbm.at[i1c], r1, sem1)
        g0.wait()
        g1.wait()
        for rr in range(CHUNK):
            def _add(j, carry, _rr=rr):
                sl = pl.ds(j * 16, 16)
                r0[_rr, sl] += r1[_rr, sl]
                return carry
            lax.fori_loop(0, HIDDEN // 16, _add, 0)
        pltpu.sync_copy(r0, out_hbm.at[pl.ds(tok0, CHUNK)])


def _combine(out_rows, idx0, idx1):
    mesh = plsc.VectorSubcoreMesh(core_axis_name="c", subcore_axis_name="s")
    f = functools.partial(
        pl.kernel,
        mesh=mesh,
        out_type=jax.ShapeDtypeStruct((NUM_TOKENS, HIDDEN), jnp.float32),
        scratch_types=[
            pltpu.VMEM((CHUNK,), jnp.int32),
            pltpu.VMEM((CHUNK,), jnp.int32),
            pltpu.VMEM((CHUNK, HIDDEN), jnp.float32),
            pltpu.VMEM((CHUNK, HIDDEN), jnp.float32),
            pltpu.SemaphoreType.DMA,
            pltpu.SemaphoreType.DMA,
        ],
    )(_combine_body)
    return f(out_rows, idx0, idx1)


def kernel(intermediate_states, w, router_logits):
    ws_sorted, idx0, idx1, b_v, e_v, lo_v, hi_v = _routing(router_logits)
    w_bf = w.astype(jnp.bfloat16)
    ws_b = jnp.broadcast_to(ws_sorted[:, None], (ROWS, 128))
    out_rows = _grouped_gemm(intermediate_states, w_bf, ws_b, b_v, e_v, lo_v, hi_v)
    return _combine(out_rows, idx0, idx1)


# speedup vs baseline: 2.3668x; 1.2454x over previous
import functools

import jax
import jax.numpy as jnp
from jax import lax
from jax.experimental import pallas as pl
from jax.experimental.pallas import tpu as pltpu
from jax.experimental.pallas import tpu_sc as plsc

NUM_TOKENS = 2048
TOPK = 2
NUM_EXPERTS = 8
HIDDEN = 1024
INTER = 4096
ROWS = NUM_TOKENS * TOPK
BLK = 128
NUM_BLOCKS = ROWS // BLK
NUM_VISITS = NUM_BLOCKS + NUM_EXPERTS - 1

NC = 2
NS = 16
NW = NC * NS
TPW = NUM_TOKENS // NW
CHUNK = 16
NCH = TPW // CHUNK


def _routing(router_logits):
    probs = jax.nn.softmax(router_logits, axis=-1)
    topk_w, topk_ids = lax.top_k(probs, TOPK)
    flat = topk_ids.reshape(-1).astype(jnp.int32)
    onehot = (flat[:, None] == jnp.arange(NUM_EXPERTS, dtype=jnp.int32)[None, :]).astype(jnp.int32)
    cum = jnp.cumsum(onehot, axis=0)
    counts = cum[-1]
    off = jnp.concatenate([jnp.zeros((1,), jnp.int32),
                           jnp.cumsum(counts).astype(jnp.int32)])
    rank = jnp.take_along_axis(cum - onehot, flat[:, None], axis=1)[:, 0]
    pos = (off[flat] + rank).astype(jnp.int32)
    ws_sorted = jnp.zeros((ROWS,), jnp.float32).at[pos].set(topk_w.reshape(-1))
    idx0 = pos[0::2]
    idx1 = pos[1::2]

    bstart = off[:-1] // BLK
    bend = -((-off[1:]) // BLK)
    nblk = jnp.where(counts > 0, bend - bstart, 0).astype(jnp.int32)
    vstart = jnp.concatenate([jnp.zeros((1,), jnp.int32),
                              jnp.cumsum(nblk).astype(jnp.int32)])
    total = vstart[-1]
    vv = jnp.arange(NUM_VISITS, dtype=jnp.int32)
    e_v = jnp.clip(jnp.searchsorted(vstart, vv, side="right").astype(jnp.int32) - 1,
                   0, NUM_EXPERTS - 1)
    b_v = bstart[e_v] + (vv - vstart[e_v])
    lo_v = jnp.maximum(off[e_v], b_v * BLK)
    hi_v = jnp.minimum(off[e_v + 1], (b_v + 1) * BLK)
    valid = vv < total
    e_pad = e_v[jnp.maximum(total - 1, 0)]
    e_v = jnp.where(valid, e_v, e_pad).astype(jnp.int32)
    b_v = jnp.where(valid, b_v, NUM_BLOCKS - 1).astype(jnp.int32)
    lo_v = jnp.where(valid, lo_v, 0).astype(jnp.int32)
    hi_v = jnp.where(valid, hi_v, 0).astype(jnp.int32)
    return ws_sorted, idx0, idx1, b_v, e_v, lo_v, hi_v


def _gemm_body(vb_ref, ve_ref, lo_ref, hi_ref, x_ref, w_ref, ws_ref, o_ref):
    v = pl.program_id(0)
    b = vb_ref[v]
    lo = lo_ref[v]
    hi = hi_ref[v]
    rows = b * BLK + lax.broadcasted_iota(jnp.int32, (BLK, 1), 0)
    mask = jnp.logical_and(rows >= lo, rows < hi)
    prod = lax.dot_general(x_ref[...], w_ref[0], (((1,), (0,)), ((), ())),
                           preferred_element_type=jnp.float32)
    wcol = ws_ref[:, 0:1]
    contrib = prod * jnp.where(mask, wcol, 0.0)
    first = jnp.logical_or(v == 0, vb_ref[jnp.maximum(v - 1, 0)] != b)

    @pl.when(first)
    def _():
        o_ref[...] = contrib

    @pl.when(jnp.logical_not(first))
    def _():
        o_ref[...] += contrib


def _grouped_gemm(x, w_bf, ws_b, b_v, e_v, lo_v, hi_v):
    grid_spec = pltpu.PrefetchScalarGridSpec(
        num_scalar_prefetch=4,
        grid=(NUM_VISITS,),
        in_specs=[
            pl.BlockSpec((BLK, INTER), lambda v, vb, ve, lo, hi: (vb[v], 0)),
            pl.BlockSpec((1, INTER, HIDDEN), lambda v, vb, ve, lo, hi: (ve[v], 0, 0)),
            pl.BlockSpec((BLK, 128), lambda v, vb, ve, lo, hi: (vb[v], 0)),
        ],
        out_specs=pl.BlockSpec((BLK, HIDDEN), lambda v, vb, ve, lo, hi: (vb[v], 0)),
    )
    return pl.pallas_call(
        _gemm_body,
        grid_spec=grid_spec,
        out_shape=jax.ShapeDtypeStruct((ROWS, HIDDEN), jnp.float32),
        compiler_params=pltpu.CompilerParams(
            dimension_semantics=("arbitrary",)),
    )(b_v, e_v, lo_v, hi_v, x, w_bf, ws_b)


def _combine_body(rows_hbm, i0_hbm, i1_hbm, out_hbm,
                  i0v, i1v, r0a, r0b, r1a, r1b,
                  s0a, s0b, s1a, s1b, soa, sob):
    wid = lax.axis_index("s") * NC + lax.axis_index("c")
    base = wid * TPW
    pltpu.sync_copy(i0_hbm.at[pl.ds(base, TPW)], i0v)
    pltpu.sync_copy(i1_hbm.at[pl.ds(base, TPW)], i1v)
    r0 = [r0a, r0b]
    r1 = [r1a, r1b]
    s0 = [s0a, s0b]
    s1 = [s1a, s1b]
    so = [soa, sob]
    copies = {}
    out_copies = {}

    def start(c):
        b = c % 2
        sl = pl.ds(c * CHUNK, CHUNK)
        copies[c] = (pltpu.async_copy(rows_hbm.at[i0v.at[sl]], r0[b], s0[b]),
                     pltpu.async_copy(rows_hbm.at[i1v.at[sl]], r1[b], s1[b]))

    start(0)
    for c in range(NCH):
        b = c % 2
        if c + 1 < NCH:
            if c >= 1:
                out_copies[c - 1].wait()
            start(c + 1)
        g0, g1 = copies.pop(c)
        g0.wait()
        g1.wait()

        def _row(rr, carry, _b=b):
            def _add(j, carry2):
                sl16 = pl.ds(j * 16, 16)
                r0[_b][rr, sl16] += r1[_b][rr, sl16]
                return carry2
            return lax.fori_loop(0, HIDDEN // 16, _add, carry, unroll=8)

        lax.fori_loop(0, CHUNK, _row, 0)
        out_copies[c] = pltpu.async_copy(
            r0[b], out_hbm.at[pl.ds(base + c * CHUNK, CHUNK)], so[b])
    out_copies[NCH - 2].wait()
    out_copies[NCH - 1].wait()


def _combine(out_rows, idx0, idx1):
    mesh = plsc.VectorSubcoreMesh(core_axis_name="c", subcore_axis_name="s")
    f = functools.partial(
        pl.kernel,
        mesh=mesh,
        out_type=jax.ShapeDtypeStruct((NUM_TOKENS, HIDDEN), jnp.float32),
        scratch_types=[
            pltpu.VMEM((TPW,), jnp.int32),
            pltpu.VMEM((TPW,), jnp.int32),
            pltpu.VMEM((CHUNK, HIDDEN), jnp.float32),
            pltpu.VMEM((CHUNK, HIDDEN), jnp.float32),
            pltpu.VMEM((CHUNK, HIDDEN), jnp.float32),
            pltpu.VMEM((CHUNK, HIDDEN), jnp.float32),
            pltpu.SemaphoreType.DMA,
            pltpu.SemaphoreType.DMA,
            pltpu.SemaphoreType.DMA,
            pltpu.SemaphoreType.DMA,
            pltpu.SemaphoreType.DMA,
            pltpu.SemaphoreType.DMA,
        ],
    )(_combine_body)
    return f(out_rows, idx0, idx1)


def kernel(intermediate_states, w, router_logits):
    ws_sorted, idx0, idx1, b_v, e_v, lo_v, hi_v = _routing(router_logits)
    ws_b = jnp.broadcast_to(ws_sorted[:, None], (ROWS, 128))
    out_rows = _grouped_gemm(intermediate_states, w, ws_b, b_v, e_v, lo_v, hi_v)
    return _combine(out_rows, idx0, idx1)


# speedup vs baseline: 2.6437x; 1.1170x over previous
import functools

import jax
import jax.numpy as jnp
from jax import lax
from jax.experimental import pallas as pl
from jax.experimental.pallas import tpu as pltpu
from jax.experimental.pallas import tpu_sc as plsc

NUM_TOKENS = 2048
TOPK = 2
NUM_EXPERTS = 8
HIDDEN = 1024
INTER = 4096
ROWS = NUM_TOKENS * TOPK
BLK = 128
NUM_BLOCKS = ROWS // BLK
NUM_VISITS = NUM_BLOCKS + NUM_EXPERTS - 1

NC = 2
NS = 16
NW = NC * NS
TPW = NUM_TOKENS // NW
CHUNK = 16
NCH = TPW // CHUNK


def _route_body(lg_ref, pos0_ref, pos1_ref, w0_ref, w1_ref, cnt_ref):
    x = lg_ref[...]
    m = jnp.max(x, axis=1, keepdims=True)
    ex = jnp.exp(x - m)
    p = ex / jnp.sum(ex, axis=1, keepdims=True)
    li = lax.broadcasted_iota(jnp.int32, (NUM_TOKENS, NUM_EXPERTS), 1)
    v0 = jnp.max(p, axis=1, keepdims=True)
    i0 = jnp.min(jnp.where(p == v0, li, NUM_EXPERTS), axis=1, keepdims=True)
    pm = jnp.where(li == i0, -jnp.inf, p)
    v1 = jnp.max(pm, axis=1, keepdims=True)
    i1 = jnp.min(jnp.where(pm == v1, li, NUM_EXPERTS), axis=1, keepdims=True)
    oh = (jnp.logical_or(li == i0, li == i1)).astype(jnp.int32)
    ohf = oh.astype(jnp.float32)
    tri = (lax.broadcasted_iota(jnp.int32, (128, 128), 1)
           < lax.broadcasted_iota(jnp.int32, (128, 128), 0)).astype(jnp.float32)
    parts = []
    run = jnp.zeros((1, NUM_EXPERTS), jnp.float32)
    for k in range(NUM_TOKENS // 128):
        blk = ohf[k * 128:(k + 1) * 128, :]
        ck = lax.dot_general(tri, blk, (((1,), (0,)), ((), ())),
                             preferred_element_type=jnp.float32)
        parts.append(ck + run)
        run = run + jnp.sum(blk, axis=0, keepdims=True)
    c_excl = jnp.concatenate(parts, axis=0).astype(jnp.int32)
    cnt_ref[...] = run.astype(jnp.int32)
    offb = jnp.sum((i0 < li).astype(jnp.int32) + (i1 < li).astype(jnp.int32),
                   axis=0, keepdims=True)
    rank0 = jnp.sum(jnp.where(li == i0, c_excl, 0), axis=1, keepdims=True)
    rank1 = jnp.sum(jnp.where(li == i1, c_excl, 0), axis=1, keepdims=True)
    off0 = jnp.sum(jnp.where(li == i0, offb, 0), axis=1, keepdims=True)
    off1 = jnp.sum(jnp.where(li == i1, offb, 0), axis=1, keepdims=True)
    pos0_ref[...] = off0 + rank0
    pos1_ref[...] = off1 + rank1
    w0_ref[...] = jnp.broadcast_to(v0, (NUM_TOKENS, 16))
    w1_ref[...] = jnp.broadcast_to(v1, (NUM_TOKENS, 16))


def _route(router_logits):
    return pl.pallas_call(
        _route_body,
        out_shape=(
            jax.ShapeDtypeStruct((NUM_TOKENS, 1), jnp.int32),
            jax.ShapeDtypeStruct((NUM_TOKENS, 1), jnp.int32),
            jax.ShapeDtypeStruct((NUM_TOKENS, 16), jnp.float32),
            jax.ShapeDtypeStruct((NUM_TOKENS, 16), jnp.float32),
            jax.ShapeDtypeStruct((1, NUM_EXPERTS), jnp.int32),
        ),
    )(router_logits)


def _routing(router_logits):
    pos0, pos1, w0b, w1b, cnt = _route(router_logits)
    idx0 = pos0.reshape(ROWS // TOPK)
    idx1 = pos1.reshape(ROWS // TOPK)
    counts = cnt.reshape(NUM_EXPERTS)
    off = jnp.concatenate([jnp.zeros((1,), jnp.int32),
                           jnp.cumsum(counts).astype(jnp.int32)])

    bstart = off[:-1] // BLK
    bend = -((-off[1:]) // BLK)
    nblk = jnp.where(counts > 0, bend - bstart, 0).astype(jnp.int32)
    vstart = jnp.concatenate([jnp.zeros((1,), jnp.int32),
                              jnp.cumsum(nblk).astype(jnp.int32)])
    total = vstart[-1]
    vv = jnp.arange(NUM_VISITS, dtype=jnp.int32)
    e_v = jnp.clip(jnp.searchsorted(vstart, vv, side="right").astype(jnp.int32) - 1,
                   0, NUM_EXPERTS - 1)
    b_v = bstart[e_v] + (vv - vstart[e_v])
    lo_v = jnp.maximum(off[e_v], b_v * BLK)
    hi_v = jnp.minimum(off[e_v + 1], (b_v + 1) * BLK)
    valid = vv < total
    e_pad = e_v[jnp.maximum(total - 1, 0)]
    e_v = jnp.where(valid, e_v, e_pad).astype(jnp.int32)
    b_v = jnp.where(valid, b_v, NUM_BLOCKS - 1).astype(jnp.int32)
    lo_v = jnp.where(valid, lo_v, 0).astype(jnp.int32)
    hi_v = jnp.where(valid, hi_v, 0).astype(jnp.int32)
    return idx0, idx1, w0b, w1b, b_v, e_v, lo_v, hi_v


def _gemm_body(vb_ref, ve_ref, lo_ref, hi_ref, x_ref, w_ref, o_ref):
    v = pl.program_id(0)
    b = vb_ref[v]
    lo = lo_ref[v]
    hi = hi_ref[v]
    rows = b * BLK + lax.broadcasted_iota(jnp.int32, (BLK, 1), 0)
    mask = jnp.logical_and(rows >= lo, rows < hi)
    prod = lax.dot_general(x_ref[...], w_ref[0], (((1,), (0,)), ((), ())),
                           preferred_element_type=jnp.float32)
    contrib = jnp.where(mask, prod, 0.0)
    first = jnp.logical_or(v == 0, vb_ref[jnp.maximum(v - 1, 0)] != b)

    @pl.when(first)
    def _():
        o_ref[...] = contrib

    @pl.when(jnp.logical_not(first))
    def _():
        o_ref[...] += contrib


def _grouped_gemm(x, w, b_v, e_v, lo_v, hi_v):
    grid_spec = pltpu.PrefetchScalarGridSpec(
        num_scalar_prefetch=4,
        grid=(NUM_VISITS,),
        in_specs=[
            pl.BlockSpec((BLK, INTER), lambda v, vb, ve, lo, hi: (vb[v], 0)),
            pl.BlockSpec((1, INTER, HIDDEN), lambda v, vb, ve, lo, hi: (ve[v], 0, 0)),
        ],
        out_specs=pl.BlockSpec((BLK, HIDDEN), lambda v, vb, ve, lo, hi: (vb[v], 0)),
    )
    return pl.pallas_call(
        _gemm_body,
        grid_spec=grid_spec,
        out_shape=jax.ShapeDtypeStruct((ROWS, HIDDEN), jnp.float32),
        compiler_params=pltpu.CompilerParams(
            dimension_semantics=("arbitrary",)),
    )(b_v, e_v, lo_v, hi_v, x, w)


def _combine_body(rows_hbm, i0_hbm, i1_hbm, w0_hbm, w1_hbm, out_hbm,
                  i0v, i1v, w0v, w1v, r0a, r0b, r1a, r1b,
                  s0a, s0b, s1a, s1b, soa, sob):
    wid = lax.axis_index("s") * NC + lax.axis_index("c")
    base = wid * TPW
    pltpu.sync_copy(i0_hbm.at[pl.ds(base, TPW)], i0v)
    pltpu.sync_copy(i1_hbm.at[pl.ds(base, TPW)], i1v)
    pltpu.sync_copy(w0_hbm.at[pl.ds(base, TPW)], w0v)
    pltpu.sync_copy(w1_hbm.at[pl.ds(base, TPW)], w1v)
    r0 = [r0a, r0b]
    r1 = [r1a, r1b]
    s0 = [s0a, s0b]
    s1 = [s1a, s1b]
    so = [soa, sob]
    copies = {}
    out_copies = {}

    def start(c):
        b = c % 2
        sl = pl.ds(c * CHUNK, CHUNK)
        copies[c] = (pltpu.async_copy(rows_hbm.at[i0v.at[sl]], r0[b], s0[b]),
                     pltpu.async_copy(rows_hbm.at[i1v.at[sl]], r1[b], s1[b]))

    start(0)
    for c in range(NCH):
        b = c % 2
        if c + 1 < NCH:
            if c >= 1:
                out_copies[c - 1].wait()
            start(c + 1)
        g0, g1 = copies.pop(c)
        g0.wait()
        g1.wait()

        def _row(rr, carry, _b=b, _c=c):
            wr0 = w0v[_c * CHUNK + rr, :]
            wr1 = w1v[_c * CHUNK + rr, :]

            def _add(j, carry2):
                sl16 = pl.ds(j * 16, 16)
                r0[_b][rr, sl16] = wr0 * r0[_b][rr, sl16] + wr1 * r1[_b][rr, sl16]
                return carry2
            return lax.fori_loop(0, HIDDEN // 16, _add, carry, unroll=8)

        lax.fori_loop(0, CHUNK, _row, 0)
        out_copies[c] = pltpu.async_copy(
            r0[b], out_hbm.at[pl.ds(base + c * CHUNK, CHUNK)], so[b])
    out_copies[NCH - 2].wait()
    out_copies[NCH - 1].wait()


def _combine(out_rows, idx0, idx1, w0b, w1b):
    mesh = plsc.VectorSubcoreMesh(core_axis_name="c", subcore_axis_name="s")
    f = functools.partial(
        pl.kernel,
        mesh=mesh,
        out_type=jax.ShapeDtypeStruct((NUM_TOKENS, HIDDEN), jnp.float32),
        scratch_types=[
            pltpu.VMEM((TPW,), jnp.int32),
            pltpu.VMEM((TPW,), jnp.int32),
            pltpu.VMEM((TPW, 16), jnp.float32),
            pltpu.VMEM((TPW, 16), jnp.float32),
            pltpu.VMEM((CHUNK, HIDDEN), jnp.float32),
            pltpu.VMEM((CHUNK, HIDDEN), jnp.float32),
            pltpu.VMEM((CHUNK, HIDDEN), jnp.float32),
            pltpu.VMEM((CHUNK, HIDDEN), jnp.float32),
            pltpu.SemaphoreType.DMA,
            pltpu.SemaphoreType.DMA,
            pltpu.SemaphoreType.DMA,
            pltpu.SemaphoreType.DMA,
            pltpu.SemaphoreType.DMA,
            pltpu.SemaphoreType.DMA,
        ],
    )(_combine_body)
    return f(out_rows, idx0, idx1, w0b, w1b)


def kernel(intermediate_states, w, router_logits):
    idx0, idx1, w0b, w1b, b_v, e_v, lo_v, hi_v = _routing(router_logits)
    out_rows = _grouped_gemm(intermediate_states, w, b_v, e_v, lo_v, hi_v)
    return _combine(out_rows, idx0, idx1, w0b, w1b)


# speedup vs baseline: 2.7342x; 1.0342x over previous
import functools

import jax
import jax.numpy as jnp
from jax import lax
from jax.experimental import pallas as pl
from jax.experimental.pallas import tpu as pltpu
from jax.experimental.pallas import tpu_sc as plsc

NUM_TOKENS = 2048
TOPK = 2
NUM_EXPERTS = 8
HIDDEN = 1024
INTER = 4096
ROWS = NUM_TOKENS * TOPK
BLK = 128
NUM_BLOCKS = ROWS // BLK
NUM_VISITS = NUM_BLOCKS + NUM_EXPERTS - 1

NC = 2
NS = 16
NW = NC * NS
TPW = NUM_TOKENS // NW
CHUNK = 16
NCH = TPW // CHUNK


def _route_body(lg_ref, pos0_ref, pos1_ref, w0_ref, w1_ref, cnt_ref):
    x = lg_ref[...]
    m = jnp.max(x, axis=1, keepdims=True)
    ex = jnp.exp(x - m)
    p = ex / jnp.sum(ex, axis=1, keepdims=True)
    li = lax.broadcasted_iota(jnp.int32, (NUM_TOKENS, NUM_EXPERTS), 1)
    v0 = jnp.max(p, axis=1, keepdims=True)
    i0 = jnp.min(jnp.where(p == v0, li, NUM_EXPERTS), axis=1, keepdims=True)
    pm = jnp.where(li == i0, -jnp.inf, p)
    v1 = jnp.max(pm, axis=1, keepdims=True)
    i1 = jnp.min(jnp.where(pm == v1, li, NUM_EXPERTS), axis=1, keepdims=True)
    oh = (jnp.logical_or(li == i0, li == i1)).astype(jnp.int32)
    ohf = oh.astype(jnp.float32)
    tri = (lax.broadcasted_iota(jnp.int32, (128, 128), 1)
           < lax.broadcasted_iota(jnp.int32, (128, 128), 0)).astype(jnp.float32)
    parts = []
    run = jnp.zeros((1, NUM_EXPERTS), jnp.float32)
    for k in range(NUM_TOKENS // 128):
        blk = ohf[k * 128:(k + 1) * 128, :]
        ck = lax.dot_general(tri, blk, (((1,), (0,)), ((), ())),
                             preferred_element_type=jnp.float32)
        parts.append(ck + run)
        run = run + jnp.sum(blk, axis=0, keepdims=True)
    c_excl = jnp.concatenate(parts, axis=0).astype(jnp.int32)
    cnt_ref[...] = run.astype(jnp.int32)
    offb = jnp.sum((i0 < li).astype(jnp.int32) + (i1 < li).astype(jnp.int32),
                   axis=0, keepdims=True)
    rank0 = jnp.sum(jnp.where(li == i0, c_excl, 0), axis=1, keepdims=True)
    rank1 = jnp.sum(jnp.where(li == i1, c_excl, 0), axis=1, keepdims=True)
    off0 = jnp.sum(jnp.where(li == i0, offb, 0), axis=1, keepdims=True)
    off1 = jnp.sum(jnp.where(li == i1, offb, 0), axis=1, keepdims=True)
    pos0_ref[...] = off0 + rank0
    pos1_ref[...] = off1 + rank1
    w0_ref[...] = jnp.broadcast_to(v0, (NUM_TOKENS, 16))
    w1_ref[...] = jnp.broadcast_to(v1, (NUM_TOKENS, 16))


def _route(router_logits):
    return pl.pallas_call(
        _route_body,
        out_shape=(
            jax.ShapeDtypeStruct((NUM_TOKENS, 1), jnp.int32),
            jax.ShapeDtypeStruct((NUM_TOKENS, 1), jnp.int32),
            jax.ShapeDtypeStruct((NUM_TOKENS, 16), jnp.float32),
            jax.ShapeDtypeStruct((NUM_TOKENS, 16), jnp.float32),
            jax.ShapeDtypeStruct((1, NUM_EXPERTS), jnp.int32),
        ),
    )(router_logits)


def _routing(router_logits):
    pos0, pos1, w0b, w1b, cnt = _route(router_logits)
    idx0 = pos0.reshape(ROWS // TOPK)
    idx1 = pos1.reshape(ROWS // TOPK)
    counts = cnt.reshape(NUM_EXPERTS)
    return idx0, idx1, w0b, w1b, counts


def _visit_sched(cnt_ref, v):
    off = jnp.int32(0)
    vst = jnp.int32(0)
    b = jnp.int32(NUM_BLOCKS - 1)
    e_sel = jnp.int32(0)
    e_last = jnp.int32(0)
    lo = jnp.int32(0)
    hi = jnp.int32(0)
    got = jnp.bool_(False)
    for e in range(NUM_EXPERTS):
        c = cnt_ref[e]
        off_e = off
        off = off + c
        bs = lax.div(off_e, jnp.int32(BLK))
        be = lax.div(off + jnp.int32(BLK - 1), jnp.int32(BLK))
        nb = jnp.where(c > 0, be - bs, 0)
        vst_e = vst
        vst = vst + nb
        sel = jnp.logical_and(v >= vst_e, v < vst)
        bv_e = bs + (v - vst_e)
        b = jnp.where(sel, bv_e, b)
        e_sel = jnp.where(sel, e, e_sel)
        e_last = jnp.where(nb > 0, e, e_last)
        lo = jnp.where(sel, jnp.maximum(off_e, bv_e * BLK), lo)
        hi = jnp.where(sel, jnp.minimum(off, (bv_e + 1) * BLK), hi)
        got = jnp.logical_or(got, sel)
    e_sel = jnp.where(got, e_sel, e_last)
    return b, e_sel, lo, hi


def _gemm_body(cnt_ref, x_ref, w_ref, o_ref):
    v = pl.program_id(0)
    b, _, lo, hi = _visit_sched(cnt_ref, v)
    bp, _, _, _ = _visit_sched(cnt_ref, jnp.maximum(v - 1, 0))
    first = jnp.logical_or(v == 0, bp != b)
    rows = b * BLK + lax.broadcasted_iota(jnp.int32, (BLK, 1), 0)
    mask = jnp.logical_and(rows >= lo, rows < hi)
    prod = lax.dot_general(x_ref[...], w_ref[0], (((1,), (0,)), ((), ())),
                           preferred_element_type=jnp.float32)
    contrib = jnp.where(mask, prod, 0.0)

    @pl.when(first)
    def _():
        o_ref[...] = contrib

    @pl.when(jnp.logical_not(first))
    def _():
        o_ref[...] += contrib


def _x_map(v, cnt):
    b, _, _, _ = _visit_sched(cnt, v)
    return b, 0


def _w_map(v, cnt):
    _, e, _, _ = _visit_sched(cnt, v)
    return e, 0, 0


def _grouped_gemm(x, w, counts):
    grid_spec = pltpu.PrefetchScalarGridSpec(
        num_scalar_prefetch=1,
        grid=(NUM_VISITS,),
        in_specs=[
            pl.BlockSpec((BLK, INTER), _x_map),
            pl.BlockSpec((1, INTER, HIDDEN), _w_map),
        ],
        out_specs=pl.BlockSpec((BLK, HIDDEN), _x_map),
    )
    return pl.pallas_call(
        _gemm_body,
        grid_spec=grid_spec,
        out_shape=jax.ShapeDtypeStruct((ROWS, HIDDEN), jnp.float32),
        compiler_params=pltpu.CompilerParams(
            dimension_semantics=("arbitrary",)),
    )(counts, x, w)


def _combine_body(rows_hbm, i0_hbm, i1_hbm, w0_hbm, w1_hbm, out_hbm,
                  i0v, i1v, w0v, w1v, r0a, r0b, r1a, r1b,
                  s0a, s0b, s1a, s1b, soa, sob):
    wid = lax.axis_index("s") * NC + lax.axis_index("c")
    base = wid * TPW
    pltpu.sync_copy(i0_hbm.at[pl.ds(base, TPW)], i0v)
    pltpu.sync_copy(i1_hbm.at[pl.ds(base, TPW)], i1v)
    pltpu.sync_copy(w0_hbm.at[pl.ds(base, TPW)], w0v)
    pltpu.sync_copy(w1_hbm.at[pl.ds(base, TPW)], w1v)
    r0 = [r0a, r0b]
    r1 = [r1a, r1b]
    s0 = [s0a, s0b]
    s1 = [s1a, s1b]
    so = [soa, sob]
    copies = {}
    out_copies = {}

    def start(c):
        b = c % 2
        sl = pl.ds(c * CHUNK, CHUNK)
        copies[c] = (pltpu.async_copy(rows_hbm.at[i0v.at[sl]], r0[b], s0[b]),
                     pltpu.async_copy(rows_hbm.at[i1v.at[sl]], r1[b], s1[b]))

    start(0)
    for c in range(NCH):
        b = c % 2
        if c + 1 < NCH:
            if c >= 1:
                out_copies[c - 1].wait()
            start(c + 1)
        g0, g1 = copies.pop(c)
        g0.wait()
        g1.wait()

        def _row(rr, carry, _b=b, _c=c):
            wr0 = w0v[_c * CHUNK + rr, :]
            wr1 = w1v[_c * CHUNK + rr, :]

            def _add(j, carry2):
                sl16 = pl.ds(j * 16, 16)
                r0[_b][rr, sl16] = wr0 * r0[_b][rr, sl16] + wr1 * r1[_b][rr, sl16]
                return carry2
            return lax.fori_loop(0, HIDDEN // 16, _add, carry, unroll=8)

        lax.fori_loop(0, CHUNK, _row, 0)
        out_copies[c] = pltpu.async_copy(
            r0[b], out_hbm.at[pl.ds(base + c * CHUNK, CHUNK)], so[b])
    out_copies[NCH - 2].wait()
    out_copies[NCH - 1].wait()


def _combine(out_rows, idx0, idx1, w0b, w1b):
    mesh = plsc.VectorSubcoreMesh(core_axis_name="c", subcore_axis_name="s")
    f = functools.partial(
        pl.kernel,
        mesh=mesh,
        out_type=jax.ShapeDtypeStruct((NUM_TOKENS, HIDDEN), jnp.float32),
        scratch_types=[
            pltpu.VMEM((TPW,), jnp.int32),
            pltpu.VMEM((TPW,), jnp.int32),
            pltpu.VMEM((TPW, 16), jnp.float32),
            pltpu.VMEM((TPW, 16), jnp.float32),
            pltpu.VMEM((CHUNK, HIDDEN), jnp.float32),
            pltpu.VMEM((CHUNK, HIDDEN), jnp.float32),
            pltpu.VMEM((CHUNK, HIDDEN), jnp.float32),
            pltpu.VMEM((CHUNK, HIDDEN), jnp.float32),
            pltpu.SemaphoreType.DMA,
            pltpu.SemaphoreType.DMA,
            pltpu.SemaphoreType.DMA,
            pltpu.SemaphoreType.DMA,
            pltpu.SemaphoreType.DMA,
            pltpu.SemaphoreType.DMA,
        ],
    )(_combine_body)
    return f(out_rows, idx0, idx1, w0b, w1b)


def kernel(intermediate_states, w, router_logits):
    idx0, idx1, w0b, w1b, counts = _routing(router_logits)
    out_rows = _grouped_gemm(intermediate_states, w, counts)
    return _combine(out_rows, idx0, idx1, w0b, w1b)


# speedup vs baseline: 2.8574x; 1.0451x over previous
import functools

import jax
import jax.numpy as jnp
from jax import lax
from jax.experimental import pallas as pl
from jax.experimental.pallas import tpu as pltpu
from jax.experimental.pallas import tpu_sc as plsc

NUM_TOKENS = 2048
TOPK = 2
NUM_EXPERTS = 8
HIDDEN = 1024
INTER = 4096
ROWS = NUM_TOKENS * TOPK
BLK = 256
NUM_BLOCKS = ROWS // BLK
NUM_VISITS = NUM_BLOCKS + NUM_EXPERTS - 1

NC = 2
NS = 16
NW = NC * NS
TPW = NUM_TOKENS // NW
CHUNK = 16
NCH = TPW // CHUNK


def _route_body(lg_ref, pos0_ref, pos1_ref, w0_ref, w1_ref, cnt_ref):
    x = lg_ref[...]
    m = jnp.max(x, axis=1, keepdims=True)
    ex = jnp.exp(x - m)
    p = ex / jnp.sum(ex, axis=1, keepdims=True)
    li = lax.broadcasted_iota(jnp.int32, (NUM_TOKENS, NUM_EXPERTS), 1)
    v0 = jnp.max(p, axis=1, keepdims=True)
    i0 = jnp.min(jnp.where(p == v0, li, NUM_EXPERTS), axis=1, keepdims=True)
    pm = jnp.where(li == i0, -jnp.inf, p)
    v1 = jnp.max(pm, axis=1, keepdims=True)
    i1 = jnp.min(jnp.where(pm == v1, li, NUM_EXPERTS), axis=1, keepdims=True)
    oh = (jnp.logical_or(li == i0, li == i1)).astype(jnp.int32)
    ohf = oh.astype(jnp.float32)
    tri = (lax.broadcasted_iota(jnp.int32, (128, 128), 1)
           < lax.broadcasted_iota(jnp.int32, (128, 128), 0)).astype(jnp.float32)
    parts = []
    run = jnp.zeros((1, NUM_EXPERTS), jnp.float32)
    for k in range(NUM_TOKENS // 128):
        blk = ohf[k * 128:(k + 1) * 128, :]
        ck = lax.dot_general(tri, blk, (((1,), (0,)), ((), ())),
                             preferred_element_type=jnp.float32)
        parts.append(ck + run)
        run = run + jnp.sum(blk, axis=0, keepdims=True)
    c_excl = jnp.concatenate(parts, axis=0).astype(jnp.int32)
    cnt_ref[...] = run.astype(jnp.int32)
    offb = jnp.sum((i0 < li).astype(jnp.int32) + (i1 < li).astype(jnp.int32),
                   axis=0, keepdims=True)
    rank0 = jnp.sum(jnp.where(li == i0, c_excl, 0), axis=1, keepdims=True)
    rank1 = jnp.sum(jnp.where(li == i1, c_excl, 0), axis=1, keepdims=True)
    off0 = jnp.sum(jnp.where(li == i0, offb, 0), axis=1, keepdims=True)
    off1 = jnp.sum(jnp.where(li == i1, offb, 0), axis=1, keepdims=True)
    pos0_ref[...] = off0 + rank0
    pos1_ref[...] = off1 + rank1
    w0_ref[...] = jnp.broadcast_to(v0, (NUM_TOKENS, 16))
    w1_ref[...] = jnp.broadcast_to(v1, (NUM_TOKENS, 16))


def _route(router_logits):
    return pl.pallas_call(
        _route_body,
        out_shape=(
            jax.ShapeDtypeStruct((NUM_TOKENS, 1), jnp.int32),
            jax.ShapeDtypeStruct((NUM_TOKENS, 1), jnp.int32),
            jax.ShapeDtypeStruct((NUM_TOKENS, 16), jnp.float32),
            jax.ShapeDtypeStruct((NUM_TOKENS, 16), jnp.float32),
            jax.ShapeDtypeStruct((1, NUM_EXPERTS), jnp.int32),
        ),
    )(router_logits)


def _routing(router_logits):
    pos0, pos1, w0b, w1b, cnt = _route(router_logits)
    idx0 = pos0.reshape(ROWS // TOPK)
    idx1 = pos1.reshape(ROWS // TOPK)
    counts = cnt.reshape(NUM_EXPERTS)
    return idx0, idx1, w0b, w1b, counts


def _visit_sched(cnt_ref, v):
    off = jnp.int32(0)
    vst = jnp.int32(0)
    b = jnp.int32(NUM_BLOCKS - 1)
    e_sel = jnp.int32(0)
    e_last = jnp.int32(0)
    lo = jnp.int32(0)
    hi = jnp.int32(0)
    got = jnp.bool_(False)
    for e in range(NUM_EXPERTS):
        c = cnt_ref[e]
        off_e = off
        off = off + c
        bs = lax.div(off_e, jnp.int32(BLK))
        be = lax.div(off + jnp.int32(BLK - 1), jnp.int32(BLK))
        nb = jnp.where(c > 0, be - bs, 0)
        vst_e = vst
        vst = vst + nb
        sel = jnp.logical_and(v >= vst_e, v < vst)
        bv_e = bs + (v - vst_e)
        b = jnp.where(sel, bv_e, b)
        e_sel = jnp.where(sel, e, e_sel)
        e_last = jnp.where(nb > 0, e, e_last)
        lo = jnp.where(sel, jnp.maximum(off_e, bv_e * BLK), lo)
        hi = jnp.where(sel, jnp.minimum(off, (bv_e + 1) * BLK), hi)
        got = jnp.logical_or(got, sel)
    e_sel = jnp.where(got, e_sel, e_last)
    return b, e_sel, lo, hi


def _gemm_body(cnt_ref, x_ref, w_ref, o_ref):
    v = pl.program_id(0)
    b, _, lo, hi = _visit_sched(cnt_ref, v)
    bp, _, _, _ = _visit_sched(cnt_ref, v - 1)
    first = jnp.logical_or(v == 0, bp != b)
    rows = b * BLK + lax.broadcasted_iota(jnp.int32, (BLK, 1), 0)
    mask = jnp.logical_and(rows >= lo, rows < hi)
    prod = lax.dot_general(x_ref[...], w_ref[0], (((1,), (0,)), ((), ())),
                           preferred_element_type=jnp.float32)
    contrib = jnp.where(mask, prod, 0.0)

    @pl.when(first)
    def _():
        o_ref[...] = contrib

    @pl.when(jnp.logical_not(first))
    def _():
        o_ref[...] += contrib


def _x_map(v, cnt):
    b, _, _, _ = _visit_sched(cnt, v)
    return b, 0


def _w_map(v, cnt):
    _, e, _, _ = _visit_sched(cnt, v)
    return e, 0, 0


def _grouped_gemm(x, w, counts):
    grid_spec = pltpu.PrefetchScalarGridSpec(
        num_scalar_prefetch=1,
        grid=(NUM_VISITS,),
        in_specs=[
            pl.BlockSpec((BLK, INTER), _x_map),
            pl.BlockSpec((1, INTER, HIDDEN), _w_map),
        ],
        out_specs=pl.BlockSpec((BLK, HIDDEN), _x_map),
    )
    return pl.pallas_call(
        _gemm_body,
        grid_spec=grid_spec,
        out_shape=jax.ShapeDtypeStruct((ROWS, HIDDEN), jnp.float32),
        compiler_params=pltpu.CompilerParams(
            dimension_semantics=("arbitrary",)),
    )(counts, x, w)


def _combine_body(rows_hbm, i0_hbm, i1_hbm, w0_hbm, w1_hbm, out_hbm,
                  i0v, i1v, w0v, w1v, r0a, r0b, r1a, r1b,
                  s0a, s0b, s1a, s1b, soa, sob):
    wid = lax.axis_index("s") * NC + lax.axis_index("c")
    base = wid * TPW
    pltpu.sync_copy(i0_hbm.at[pl.ds(base, TPW)], i0v)
    pltpu.sync_copy(i1_hbm.at[pl.ds(base, TPW)], i1v)
    pltpu.sync_copy(w0_hbm.at[pl.ds(base, TPW)], w0v)
    pltpu.sync_copy(w1_hbm.at[pl.ds(base, TPW)], w1v)
    r0 = [r0a, r0b]
    r1 = [r1a, r1b]
    s0 = [s0a, s0b]
    s1 = [s1a, s1b]
    so = [soa, sob]
    copies = {}
    out_copies = {}

    def start(c):
        b = c % 2
        sl = pl.ds(c * CHUNK, CHUNK)
        copies[c] = (pltpu.async_copy(rows_hbm.at[i0v.at[sl]], r0[b], s0[b]),
                     pltpu.async_copy(rows_hbm.at[i1v.at[sl]], r1[b], s1[b]))

    start(0)
    for c in range(NCH):
        b = c % 2
        if c + 1 < NCH:
            if c >= 1:
                out_copies[c - 1].wait()
            start(c + 1)
        g0, g1 = copies.pop(c)
        g0.wait()
        g1.wait()

        def _row(rr, carry, _b=b, _c=c):
            wr0 = w0v[_c * CHUNK + rr, :]
            wr1 = w1v[_c * CHUNK + rr, :]

            def _add(j, carry2):
                sl16 = pl.ds(j * 16, 16)
                r0[_b][rr, sl16] = wr0 * r0[_b][rr, sl16] + wr1 * r1[_b][rr, sl16]
                return carry2
            return lax.fori_loop(0, HIDDEN // 16, _add, carry, unroll=8)

        lax.fori_loop(0, CHUNK, _row, 0)
        out_copies[c] = pltpu.async_copy(
            r0[b], out_hbm.at[pl.ds(base + c * CHUNK, CHUNK)], so[b])
    out_copies[NCH - 2].wait()
    out_copies[NCH - 1].wait()


def _combine(out_rows, idx0, idx1, w0b, w1b):
    mesh = plsc.VectorSubcoreMesh(core_axis_name="c", subcore_axis_name="s")
    f = functools.partial(
        pl.kernel,
        mesh=mesh,
        out_type=jax.ShapeDtypeStruct((NUM_TOKENS, HIDDEN), jnp.float32),
        scratch_types=[
            pltpu.VMEM((TPW,), jnp.int32),
            pltpu.VMEM((TPW,), jnp.int32),
            pltpu.VMEM((TPW, 16), jnp.float32),
            pltpu.VMEM((TPW, 16), jnp.float32),
            pltpu.VMEM((CHUNK, HIDDEN), jnp.float32),
            pltpu.VMEM((CHUNK, HIDDEN), jnp.float32),
            pltpu.VMEM((CHUNK, HIDDEN), jnp.float32),
            pltpu.VMEM((CHUNK, HIDDEN), jnp.float32),
            pltpu.SemaphoreType.DMA,
            pltpu.SemaphoreType.DMA,
            pltpu.SemaphoreType.DMA,
            pltpu.SemaphoreType.DMA,
            pltpu.SemaphoreType.DMA,
            pltpu.SemaphoreType.DMA,
        ],
    )(_combine_body)
    return f(out_rows, idx0, idx1, w0b, w1b)


def kernel(intermediate_states, w, router_logits):
    idx0, idx1, w0b, w1b, counts = _routing(router_logits)
    out_rows = _grouped_gemm(intermediate_states, w, counts)
    return _combine(out_rows, idx0, idx1, w0b, w1b)
